# Initial kernel scaffold; baseline (speedup 1.0000x reference)
import functools

import jax
import jax.numpy as jnp
from jax import lax
from jax.experimental import pallas as pl
from jax.experimental.pallas import tpu as pltpu
from jax.experimental.pallas import tpu_sc as plsc

N = 10000
E = 320000
F_IN = 128
H = 128
C = 64

NC = 2
NS = 16
LANES = 128

N_PAD = 10240
ROWS_TOTAL = 2560
E_PAD = ROWS_TOTAL * LANES
ROWS_PER_SUB = ROWS_TOTAL // NS
NODE_ROWS_PER_SUB = N_PAD // NS

_MESH = plsc.VectorSubcoreMesh(core_axis_name="c", subcore_axis_name="s")


def _deg_body(dst_hbm, ew_hbm, zeros_hbm, out_hbm, dst_v, ew_v, acc):
    c = lax.axis_index("c")
    s = lax.axis_index("s")
    pltpu.sync_copy(zeros_hbm, acc.at[pl.ds(s * NODE_ROWS_PER_SUB, NODE_ROWS_PER_SUB)])
    plsc.subcore_barrier()
    r0 = s * ROWS_PER_SUB
    pltpu.sync_copy(dst_hbm.at[c, pl.ds(r0, ROWS_PER_SUB)], dst_v)
    pltpu.sync_copy(ew_hbm.at[c, pl.ds(r0, ROWS_PER_SUB)], ew_v)

    @pl.loop(0, ROWS_PER_SUB)
    def _(j):
        pltpu.sync_copy(ew_v.at[j], acc.at[dst_v.at[j]], add=True)

    plsc.subcore_barrier()
    sl = pl.ds(s * NODE_ROWS_PER_SUB, NODE_ROWS_PER_SUB)
    pltpu.sync_copy(acc.at[sl], out_hbm.at[c, sl])


_deg_call = pl.kernel(
    _deg_body,
    out_type=jax.ShapeDtypeStruct((NC, N_PAD), jnp.float32),
    mesh=_MESH,
    scratch_types=[
        pltpu.VMEM((ROWS_PER_SUB, LANES), jnp.int32),
        pltpu.VMEM((ROWS_PER_SUB, LANES), jnp.float32),
        pltpu.VMEM_SHARED((N_PAD,), jnp.float32),
    ],
)


def _make_conv(F, stacked):
    KR = 4
    K = KR * LANES
    CH = ROWS_PER_SUB // KR

    def body(table_hbm, src_hbm, dst_hbm, ew_hbm, dinv_hbm, zeros_hbm, out_hbm,
             src_v, dst_v, ew_v, rows_v, dinv_v, acc, sem):
        c = lax.axis_index("c")
        s = lax.axis_index("s")
        pltpu.sync_copy(dinv_hbm.at[c], dinv_v)
        pltpu.sync_copy(zeros_hbm, acc.at[pl.ds(s * NODE_ROWS_PER_SUB, NODE_ROWS_PER_SUB), :])
        plsc.subcore_barrier()
        r0 = s * ROWS_PER_SUB
        off = c * N

        @pl.loop(0, CH)
        def _(ci):
            r = r0 + ci * KR
            pltpu.sync_copy(src_hbm.at[c, pl.ds(r, KR)], src_v)
            pltpu.sync_copy(dst_hbm.at[c, pl.ds(r, KR)], dst_v)
            pltpu.sync_copy(ew_hbm.at[c, pl.ds(r, KR)], ew_v)

            @pl.loop(0, KR)
            def _(j):
                for q in range(LANES // 16):
                    sl = (j, pl.ds(q * 16, 16))
                    idx = src_v[sl]
                    ew_v[sl] = ew_v[sl] * plsc.load_gather(dinv_v, [idx])
                    if stacked:
                        src_v[sl] = idx + off

            for j in range(KR):
                pltpu.async_copy(table_hbm.at[src_v.at[j]],
                                 rows_v.at[pl.ds(j * LANES, LANES), :], sem).wait()

            @pl.loop(0, KR)
            def _(j):
                @pl.loop(0, LANES)
                def _(l):
                    i = j * LANES + l
                    sv = plsc.load_gather(
                        ew_v, [jnp.full((16,), j, jnp.int32),
                               jnp.full((16,), l, jnp.int32)])
                    for f in range(F // 16):
                        rows_v[i, pl.ds(f * 16, 16)] = rows_v[i, pl.ds(f * 16, 16)] * sv

            for j in range(KR):
                pltpu.sync_copy(rows_v.at[pl.ds(j * LANES, LANES), :],
                                acc.at[dst_v.at[j]], add=True)

        plsc.subcore_barrier()
        sl = pl.ds(s * NODE_ROWS_PER_SUB, NODE_ROWS_PER_SUB)
        pltpu.sync_copy(acc.at[sl, :], out_hbm.at[c, sl, :])

    return pl.kernel(
        body,
        out_type=jax.ShapeDtypeStruct((NC, N_PAD, F), jnp.float32),
        mesh=_MESH,
        scratch_types=[
            pltpu.VMEM((KR, LANES), jnp.int32),
            pltpu.VMEM((KR, LANES), jnp.int32),
            pltpu.VMEM((KR, LANES), jnp.float32),
            pltpu.VMEM((K, F), jnp.float32),
            pltpu.VMEM((N_PAD,), jnp.float32),
            pltpu.VMEM_SHARED((N_PAD, F), jnp.float32),
            pltpu.SemaphoreType.DMA,
        ],
    )


_conv_l1 = _make_conv(H, stacked=False)
_conv_l2 = _make_conv(C, stacked=True)


def _dot(a, b):
    return lax.dot_general(a, b, (((1,), (0,)), ((), ())),
                           preferred_element_type=jnp.float32,
                           precision=lax.Precision.HIGHEST)


def _mm_body(x_ref, w_ref, o_ref):
    o_ref[...] = _dot(x_ref[...], w_ref[...])


_mm_call = pl.pallas_call(
    _mm_body, out_shape=jax.ShapeDtypeStruct((N, H), jnp.float32))


def _dinv_body(deg_ref, o_ref):
    o_ref[...] = lax.rsqrt(deg_ref[...] + 1.0)


_dinv_call = pl.pallas_call(
    _dinv_body, out_shape=jax.ShapeDtypeStruct((NC, N_PAD), jnp.float32))


def _layer1_post_body(acc_ref, h1_ref, dinv_ref, b1_ref, w2_ref, o_ref):
    h1 = h1_ref[...]
    for k in range(2):
        d = dinv_ref[k, :N]
        t = acc_ref[k, :N, :] * d[:, None] + h1 * (d * d)[:, None] + b1_ref[...][None, :]
        o_ref[k, :, :] = _dot(jnp.maximum(t, 0.0), w2_ref[...])


_layer1_post = pl.pallas_call(
    _layer1_post_body, out_shape=jax.ShapeDtypeStruct((2, N, C), jnp.float32))


def _final_body(acc_ref, h2_ref, dinv_ref, b2_ref, fcw_ref, fcb_ref,
                out_ref, x1_ref, x2_ref):
    xs = []
    for k in range(2):
        d = dinv_ref[k, :N]
        t = acc_ref[k, :N, :] * d[:, None] + h2_ref[k] * (d * d)[:, None] \
            + b2_ref[...][None, :]
        xs.append(t)
    x1_ref[...] = xs[0]
    x2_ref[...] = xs[1]
    y = _dot(jnp.concatenate(xs, axis=1), fcw_ref[...]) + fcb_ref[...][None, :]
    m = jnp.max(y, axis=1, keepdims=True)
    z = y - m
    out_ref[...] = z - jnp.log(jnp.sum(jnp.exp(z), axis=1, keepdims=True))


_final_call = pl.pallas_call(
    _final_body,
    out_shape=(
        jax.ShapeDtypeStruct((N, C), jnp.float32),
        jax.ShapeDtypeStruct((N, C), jnp.float32),
        jax.ShapeDtypeStruct((N, C), jnp.float32),
    ))


def _pack_edges(edge_index1, edge_index2, edge_weight1, edge_weight2):
    pad = E_PAD - E
    pad_idx = jnp.arange(pad, dtype=jnp.int32) * 37 % N

    def cat(a, fill):
        return jnp.concatenate([a, fill])

    src = jnp.stack([cat(edge_index1[0], pad_idx), cat(edge_index2[0], pad_idx)])
    dst = jnp.stack([cat(edge_index1[1], pad_idx), cat(edge_index2[1], pad_idx)])
    zz = jnp.zeros((pad,), jnp.float32)
    ew = jnp.stack([cat(edge_weight1, zz), cat(edge_weight2, zz)])
    shp = (2, ROWS_TOTAL, LANES)
    return src.reshape(shp), dst.reshape(shp), ew.reshape(shp)


def kernel(x, edge_index1, edge_index2, edge_weight1, edge_weight2,
           W1, b1, W2, b2, fc_W, fc_b):
    src, dst, ew = _pack_edges(edge_index1, edge_index2, edge_weight1, edge_weight2)
    zeros_deg = jnp.zeros((NODE_ROWS_PER_SUB,), jnp.float32)
    zeros_l1 = jnp.zeros((NODE_ROWS_PER_SUB, H), jnp.float32)
    zeros_l2 = jnp.zeros((NODE_ROWS_PER_SUB, C), jnp.float32)

    deg = _deg_call(dst, ew, zeros_deg)
    h1 = _mm_call(x, W1)
    dinv = _dinv_call(deg)
    acc1 = _conv_l1(h1, src, dst, ew, dinv, zeros_l1)
    h2 = _layer1_post(acc1, h1, dinv, b1, W2)
    acc2 = _conv_l2(h2.reshape(2 * N, C), src, dst, ew, dinv, zeros_l2)
    out, x_1, x_2 = _final_call(acc2, h2, dinv, b2, fc_W, fc_b)
    return (out, x_1, x_2)


# speedup vs baseline: 11.4580x; 11.4580x over previous
import dataclasses
import functools

import jax
import jax.numpy as jnp
from jax import lax
from jax.experimental import pallas as pl
from jax.experimental.pallas import tpu as pltpu
from jax.experimental.pallas import tpu_sc as plsc

N = 10000
E = 320000
F_IN = 128
H = 128
C = 64

NC = 2
NS = 16
LANES = 128

N_PAD = 10240
ROWS_TOTAL = 2560
E_PAD = ROWS_TOTAL * LANES
ROWS_PER_SUB = ROWS_TOTAL // NS
NODE_ROWS_PER_SUB = N_PAD // NS

_MESH = plsc.VectorSubcoreMesh(core_axis_name="c", subcore_axis_name="s")

_SC_PARAMS = pltpu.CompilerParams()
if "needs_layout_passes" in pltpu.CompilerParams.__dataclass_fields__:
    _SC_PARAMS = dataclasses.replace(
        _SC_PARAMS, needs_layout_passes=False, use_tc_tiling_on_sc=False)


def _deg_body(dst_hbm, ew_hbm, zeros_hbm, out_hbm, dst_v, ew_v, acc):
    c = lax.axis_index("c")
    s = lax.axis_index("s")
    pltpu.sync_copy(zeros_hbm, acc.at[pl.ds(s * NODE_ROWS_PER_SUB, NODE_ROWS_PER_SUB)])
    plsc.subcore_barrier()
    r0 = s * ROWS_PER_SUB
    pltpu.sync_copy(dst_hbm.at[c, pl.ds(r0, ROWS_PER_SUB)], dst_v)
    pltpu.sync_copy(ew_hbm.at[c, pl.ds(r0, ROWS_PER_SUB)], ew_v)

    @pl.loop(0, ROWS_PER_SUB)
    def _(j):
        pltpu.sync_copy(ew_v.at[j], acc.at[dst_v.at[j]], add=True)

    plsc.subcore_barrier()
    sl = pl.ds(s * NODE_ROWS_PER_SUB, NODE_ROWS_PER_SUB)
    pltpu.sync_copy(acc.at[sl], out_hbm.at[c, sl])


_deg_call = pl.kernel(
    _deg_body,
    out_type=jax.ShapeDtypeStruct((NC, N_PAD), jnp.float32),
    mesh=_MESH,
    scratch_types=[
        pltpu.VMEM((ROWS_PER_SUB, LANES), jnp.int32),
        pltpu.VMEM((ROWS_PER_SUB, LANES), jnp.float32),
        pltpu.VMEM_SHARED((N_PAD,), jnp.float32),
    ],
)


def _make_conv(F, stacked):
    KR = 4
    K = KR * LANES
    CH = ROWS_PER_SUB // KR

    def body(table_hbm, src_hbm, dst_hbm, ew_hbm, dinv_hbm, zeros_hbm, out_hbm,
             src_v, dst_v, ew_v, rows_v, dinv_v, acc, sem):
        c = lax.axis_index("c")
        s = lax.axis_index("s")
        pltpu.sync_copy(dinv_hbm.at[c], dinv_v)
        pltpu.sync_copy(zeros_hbm, acc.at[pl.ds(s * NODE_ROWS_PER_SUB, NODE_ROWS_PER_SUB), :])
        plsc.subcore_barrier()
        r0 = s * ROWS_PER_SUB
        off = c * N

        @pl.loop(0, CH)
        def _(ci):
            r = r0 + ci * KR
            pltpu.sync_copy(src_hbm.at[c, pl.ds(r, KR)], src_v)
            pltpu.sync_copy(dst_hbm.at[c, pl.ds(r, KR)], dst_v)
            pltpu.sync_copy(ew_hbm.at[c, pl.ds(r, KR)], ew_v)

            @pl.loop(0, KR)
            def _(j):
                for q in range(LANES // 16):
                    sl = (j, pl.ds(q * 16, 16))
                    idx = src_v[sl]
                    ew_v[sl] = ew_v[sl] * plsc.load_gather(dinv_v, [idx])
                    if stacked:
                        src_v[sl] = idx + off

            for j in range(KR):
                pltpu.async_copy(table_hbm.at[src_v.at[j]],
                                 rows_v.at[pl.ds(j * LANES, LANES), :], sem).wait()

            @pl.loop(0, KR)
            def _(j):
                @pl.loop(0, LANES)
                def _(l):
                    i = j * LANES + l
                    sv = plsc.load_gather(
                        ew_v, [jnp.full((16,), j, jnp.int32),
                               jnp.full((16,), l, jnp.int32)])
                    for f in range(F // 16):
                        rows_v[i, pl.ds(f * 16, 16)] = rows_v[i, pl.ds(f * 16, 16)] * sv

            for j in range(KR):
                pltpu.sync_copy(rows_v.at[pl.ds(j * LANES, LANES), :],
                                acc.at[dst_v.at[j]], add=True)

        plsc.subcore_barrier()
        sl = pl.ds(s * NODE_ROWS_PER_SUB, NODE_ROWS_PER_SUB)
        pltpu.sync_copy(acc.at[sl, :], out_hbm.at[c, sl, :])

    return pl.kernel(
        body,
        out_type=jax.ShapeDtypeStruct((NC, N_PAD, F), jnp.float32),
        mesh=_MESH,
        scratch_types=[
            pltpu.VMEM((KR, LANES), jnp.int32),
            pltpu.VMEM((KR, LANES), jnp.int32),
            pltpu.VMEM((KR, LANES), jnp.float32),
            pltpu.VMEM((K, F), jnp.float32),
            pltpu.VMEM((N_PAD,), jnp.float32),
            pltpu.VMEM_SHARED((N_PAD, F), jnp.float32),
            pltpu.SemaphoreType.DMA,
        ],
        compiler_params=_SC_PARAMS,
    )


_conv_half = _make_conv(C, stacked=False)
_conv_l2 = _make_conv(C, stacked=True)


def _dot(a, b):
    return lax.dot_general(a, b, (((1,), (0,)), ((), ())),
                           preferred_element_type=jnp.float32,
                           precision=lax.Precision.HIGHEST)


def _mm_body(x_ref, w_ref, o_ref):
    o_ref[...] = _dot(x_ref[...], w_ref[...])


_mm_call = pl.pallas_call(
    _mm_body, out_shape=jax.ShapeDtypeStruct((N, H), jnp.float32))


def _dinv_body(deg_ref, o_ref):
    o_ref[...] = lax.rsqrt(deg_ref[...] + 1.0)


_dinv_call = pl.pallas_call(
    _dinv_body, out_shape=jax.ShapeDtypeStruct((NC, N_PAD), jnp.float32))


def _layer1_post_body(acca_ref, accb_ref, h1_ref, dinv_ref, b1_ref, w2_ref, o_ref):
    d = dinv_ref[0, 0, :N]
    a = jnp.concatenate([acca_ref[0, :N, :], accb_ref[0, :N, :]], axis=1)
    t = a * d[:, None] + h1_ref[...] * (d * d)[:, None] + b1_ref[...][None, :]
    o_ref[0, :, :] = _dot(jnp.maximum(t, 0.0), w2_ref[...])


_layer1_post = pl.pallas_call(
    _layer1_post_body,
    grid=(2,),
    in_specs=[
        pl.BlockSpec((1, N_PAD, C), lambda k: (k, 0, 0)),
        pl.BlockSpec((1, N_PAD, C), lambda k: (k, 0, 0)),
        pl.BlockSpec((N, H), lambda k: (0, 0)),
        pl.BlockSpec((1, 1, N_PAD), lambda k: (k, 0, 0)),
        pl.BlockSpec((H,), lambda k: (0,)),
        pl.BlockSpec((H, C), lambda k: (0, 0)),
    ],
    out_specs=pl.BlockSpec((1, N, C), lambda k: (k, 0, 0)),
    out_shape=jax.ShapeDtypeStruct((2, N, C), jnp.float32))


def _final_body(acc_ref, h2_ref, dinv_ref, b2_ref, fcw_ref, fcb_ref,
                out_ref, x1_ref, x2_ref):
    xs = []
    for k in range(2):
        d = dinv_ref[k, :N]
        t = acc_ref[k, :N, :] * d[:, None] + h2_ref[k] * (d * d)[:, None] \
            + b2_ref[...][None, :]
        xs.append(t)
    x1_ref[...] = xs[0]
    x2_ref[...] = xs[1]
    y = _dot(jnp.concatenate(xs, axis=1), fcw_ref[...]) + fcb_ref[...][None, :]
    m = jnp.max(y, axis=1, keepdims=True)
    z = y - m
    out_ref[...] = z - jnp.log(jnp.sum(jnp.exp(z), axis=1, keepdims=True))


_final_call = pl.pallas_call(
    _final_body,
    out_shape=(
        jax.ShapeDtypeStruct((N, C), jnp.float32),
        jax.ShapeDtypeStruct((N, C), jnp.float32),
        jax.ShapeDtypeStruct((N, C), jnp.float32),
    ))


def _pack_edges(edge_index1, edge_index2, edge_weight1, edge_weight2):
    pad = E_PAD - E
    pad_idx = jnp.arange(pad, dtype=jnp.int32) * 37 % N

    def cat(a, fill):
        return jnp.concatenate([a, fill])

    src = jnp.stack([cat(edge_index1[0], pad_idx), cat(edge_index2[0], pad_idx)])
    dst = jnp.stack([cat(edge_index1[1], pad_idx), cat(edge_index2[1], pad_idx)])
    zz = jnp.zeros((pad,), jnp.float32)
    ew = jnp.stack([cat(edge_weight1, zz), cat(edge_weight2, zz)])
    shp = (2, ROWS_TOTAL, LANES)
    return src.reshape(shp), dst.reshape(shp), ew.reshape(shp)


def kernel(x, edge_index1, edge_index2, edge_weight1, edge_weight2,
           W1, b1, W2, b2, fc_W, fc_b):
    src, dst, ew = _pack_edges(edge_index1, edge_index2, edge_weight1, edge_weight2)
    zeros_deg = jnp.zeros((NODE_ROWS_PER_SUB,), jnp.float32)
    zeros_c = jnp.zeros((NODE_ROWS_PER_SUB, C), jnp.float32)

    deg = _deg_call(dst, ew, zeros_deg)
    h1 = _mm_call(x, W1)
    dinv = _dinv_call(deg)
    acc1a = _conv_half(h1[:, :C], src, dst, ew, dinv, zeros_c)
    acc1b = _conv_half(h1[:, C:], src, dst, ew, dinv, zeros_c)
    h2 = _layer1_post(acc1a, acc1b, h1, dinv.reshape(2, 1, N_PAD), b1, W2)
    acc2 = _conv_l2(h2.reshape(2 * N, C), src, dst, ew, dinv, zeros_c)
    out, x_1, x_2 = _final_call(acc2, h2, dinv, b2, fc_W, fc_b)
    return (out, x_1, x_2)


# speedup vs baseline: 20.1673x; 1.7601x over previous
import dataclasses
import functools

import jax
import jax.numpy as jnp
from jax import lax
from jax.experimental import pallas as pl
from jax.experimental.pallas import tpu as pltpu
from jax.experimental.pallas import tpu_sc as plsc

N = 10000
E = 320000
F_IN = 128
H = 128
C = 64

NC = 2
NS = 16
LANES = 128

N_PAD = 10240
ROWS_TOTAL = 2560
E_PAD = ROWS_TOTAL * LANES
ROWS_PER_SUB = ROWS_TOTAL // NS
NODE_ROWS_PER_SUB = N_PAD // NS

_MESH = plsc.VectorSubcoreMesh(core_axis_name="c", subcore_axis_name="s")

_SC_PARAMS = pltpu.CompilerParams()
if "needs_layout_passes" in pltpu.CompilerParams.__dataclass_fields__:
    _SC_PARAMS = dataclasses.replace(
        _SC_PARAMS, needs_layout_passes=False, use_tc_tiling_on_sc=False)


def _deg_body(dst_hbm, ew_hbm, zeros_hbm, out_hbm, dst_v, ew_v, acc):
    c = lax.axis_index("c")
    s = lax.axis_index("s")
    pltpu.sync_copy(zeros_hbm, acc.at[pl.ds(s * NODE_ROWS_PER_SUB, NODE_ROWS_PER_SUB)])
    plsc.subcore_barrier()
    r0 = s * ROWS_PER_SUB
    pltpu.sync_copy(dst_hbm.at[c, pl.ds(r0, ROWS_PER_SUB)], dst_v)
    pltpu.sync_copy(ew_hbm.at[c, pl.ds(r0, ROWS_PER_SUB)], ew_v)

    @pl.loop(0, ROWS_PER_SUB)
    def _(j):
        pltpu.sync_copy(ew_v.at[j], acc.at[dst_v.at[j]], add=True)

    plsc.subcore_barrier()
    sl = pl.ds(s * NODE_ROWS_PER_SUB, NODE_ROWS_PER_SUB)
    pltpu.sync_copy(acc.at[sl], out_hbm.at[c, sl])


_deg_call = pl.kernel(
    _deg_body,
    out_type=jax.ShapeDtypeStruct((NC, N_PAD), jnp.float32),
    mesh=_MESH,
    scratch_types=[
        pltpu.VMEM((ROWS_PER_SUB, LANES), jnp.int32),
        pltpu.VMEM((ROWS_PER_SUB, LANES), jnp.float32),
        pltpu.VMEM_SHARED((N_PAD,), jnp.float32),
    ],
)


def _make_conv(F, stacked):
    KR = 4
    K = KR * LANES
    CH = ROWS_PER_SUB // KR

    def body(table_hbm, src_hbm, dst_hbm, ew_hbm, dinv_hbm, zeros_hbm, out_hbm,
             dinv_v, rows0, rows1, s0, s1, s2, s3, d0, d1, d2, d3,
             e0, e1, e2, e3, acc, sem_i, sem_g0, sem_g1, sem_s):
        c = lax.axis_index("c")
        s = lax.axis_index("s")
        srcb = [s0, s1, s2, s3]
        dstb = [d0, d1, d2, d3]
        ewb = [e0, e1, e2, e3]
        rowsb = [rows0, rows1]
        semg = [sem_g0, sem_g1]
        pltpu.sync_copy(dinv_hbm.at[c], dinv_v)
        pltpu.sync_copy(zeros_hbm, acc.at[pl.ds(s * NODE_ROWS_PER_SUB, NODE_ROWS_PER_SUB), :])
        plsc.subcore_barrier()
        r0 = s * ROWS_PER_SUB
        off = c * N

        def fire_idx(cc, b):
            r = r0 + cc * KR
            pltpu.async_copy(src_hbm.at[c, pl.ds(r, KR)], srcb[b], sem_i)
            pltpu.async_copy(dst_hbm.at[c, pl.ds(r, KR)], dstb[b], sem_i)
            pltpu.async_copy(ew_hbm.at[c, pl.ds(r, KR)], ewb[b], sem_i)

        def wait_idx(cc, b):
            r = r0 + cc * KR
            pltpu.make_async_copy(src_hbm.at[c, pl.ds(r, KR)], srcb[b], sem_i).wait()
            pltpu.make_async_copy(dst_hbm.at[c, pl.ds(r, KR)], dstb[b], sem_i).wait()
            pltpu.make_async_copy(ew_hbm.at[c, pl.ds(r, KR)], ewb[b], sem_i).wait()

        def scompute(b):
            sb, eb = srcb[b], ewb[b]

            @pl.loop(0, KR)
            def _(j):
                for q in range(LANES // 16):
                    sl = (j, pl.ds(q * 16, 16))
                    idx = sb[sl]
                    eb[sl] = eb[sl] * plsc.load_gather(dinv_v, [idx])
                    if stacked:
                        sb[sl] = idx + off

        def fire_gather(b4, b2):
            for j in range(KR):
                pltpu.async_copy(table_hbm.at[srcb[b4].at[j]],
                                 rowsb[b2].at[pl.ds(j * LANES, LANES), :], semg[b2])

        def wait_gather(b4, b2):
            for j in range(KR):
                pltpu.make_async_copy(table_hbm.at[srcb[b4].at[j]],
                                      rowsb[b2].at[pl.ds(j * LANES, LANES), :],
                                      semg[b2]).wait()

        def scale(b4, b2):
            rb, eb = rowsb[b2], ewb[b4]

            @pl.loop(0, KR)
            def _(j):
                jv = jnp.full((16,), j, jnp.int32)

                @pl.loop(0, LANES)
                def _(l):
                    sv = plsc.load_gather(eb, [jv, jnp.full((16,), l, jnp.int32)])
                    i = j * LANES + l
                    for f in range(F // 16):
                        rb[i, pl.ds(f * 16, 16)] = rb[i, pl.ds(f * 16, 16)] * sv

        def fire_scatter(b4, b2):
            for j in range(KR):
                pltpu.async_copy(rowsb[b2].at[pl.ds(j * LANES, LANES), :],
                                 acc.at[dstb[b4].at[j]], sem_s, add=True)

        def wait_scatter(b4, b2):
            for j in range(KR):
                pltpu.make_async_copy(rowsb[b2].at[pl.ds(j * LANES, LANES), :],
                                      acc.at[dstb[b4].at[j]], sem_s).wait()

        fire_idx(0, 0)
        wait_idx(0, 0)
        scompute(0)
        fire_gather(0, 0)
        fire_idx(1, 1)
        wait_idx(1, 1)
        scompute(1)
        fire_idx(2, 2)

        @pl.loop(0, CH, step=4)
        def _(ci):
            for k in range(4):
                cc = ci + k
                b4 = k % 4
                b2 = k % 2

                @pl.when(cc >= 1)
                def _():
                    wait_scatter((k - 1) % 4, (k - 1) % 2)

                @pl.when(cc + 1 < CH)
                def _():
                    fire_gather((k + 1) % 4, (k + 1) % 2)

                @pl.when(cc + 2 < CH)
                def _():
                    wait_idx(cc + 2, (k + 2) % 4)
                    scompute((k + 2) % 4)

                @pl.when(cc + 3 < CH)
                def _():
                    fire_idx(cc + 3, (k + 3) % 4)

                wait_gather(b4, b2)
                scale(b4, b2)
                fire_scatter(b4, b2)

        wait_scatter((CH - 1) % 4, (CH - 1) % 2)
        plsc.subcore_barrier()
        sl = pl.ds(s * NODE_ROWS_PER_SUB, NODE_ROWS_PER_SUB)
        pltpu.sync_copy(acc.at[sl, :], out_hbm.at[c, sl, :])

    idx_bufs = ([pltpu.VMEM((KR, LANES), jnp.int32)] * 8
                + [pltpu.VMEM((KR, LANES), jnp.float32)] * 4)
    return pl.kernel(
        body,
        out_type=jax.ShapeDtypeStruct((NC, N_PAD, F), jnp.float32),
        mesh=_MESH,
        scratch_types=[
            pltpu.VMEM((N_PAD,), jnp.float32),
            pltpu.VMEM((K, F), jnp.float32),
            pltpu.VMEM((K, F), jnp.float32),
            *idx_bufs,
            pltpu.VMEM_SHARED((N_PAD, F), jnp.float32),
            pltpu.SemaphoreType.DMA,
            pltpu.SemaphoreType.DMA,
            pltpu.SemaphoreType.DMA,
            pltpu.SemaphoreType.DMA,
        ],
        compiler_params=_SC_PARAMS,
    )


_conv_half = _make_conv(C, stacked=False)
_conv_l2 = _make_conv(C, stacked=True)


def _dot(a, b):
    return lax.dot_general(a, b, (((1,), (0,)), ((), ())),
                           preferred_element_type=jnp.float32,
                           precision=lax.Precision.HIGHEST)


def _mm_body(x_ref, w_ref, o_ref):
    o_ref[...] = _dot(x_ref[...], w_ref[...])


_mm_call = pl.pallas_call(
    _mm_body, out_shape=jax.ShapeDtypeStruct((N, H), jnp.float32))


def _dinv_body(deg_ref, o_ref):
    o_ref[...] = lax.rsqrt(deg_ref[...] + 1.0)


_dinv_call = pl.pallas_call(
    _dinv_body, out_shape=jax.ShapeDtypeStruct((NC, N_PAD), jnp.float32))


def _layer1_post_body(acca_ref, accb_ref, h1_ref, dinv_ref, b1_ref, w2_ref, o_ref):
    d = dinv_ref[0, 0, :N]
    a = jnp.concatenate([acca_ref[0, :N, :], accb_ref[0, :N, :]], axis=1)
    t = a * d[:, None] + h1_ref[...] * (d * d)[:, None] + b1_ref[...][None, :]
    o_ref[0, :, :] = _dot(jnp.maximum(t, 0.0), w2_ref[...])


_layer1_post = pl.pallas_call(
    _layer1_post_body,
    grid=(2,),
    in_specs=[
        pl.BlockSpec((1, N_PAD, C), lambda k: (k, 0, 0)),
        pl.BlockSpec((1, N_PAD, C), lambda k: (k, 0, 0)),
        pl.BlockSpec((N, H), lambda k: (0, 0)),
        pl.BlockSpec((1, 1, N_PAD), lambda k: (k, 0, 0)),
        pl.BlockSpec((H,), lambda k: (0,)),
        pl.BlockSpec((H, C), lambda k: (0, 0)),
    ],
    out_specs=pl.BlockSpec((1, N, C), lambda k: (k, 0, 0)),
    out_shape=jax.ShapeDtypeStruct((2, N, C), jnp.float32))


def _final_body(acc_ref, h2_ref, dinv_ref, b2_ref, fcw_ref, fcb_ref,
                out_ref, x1_ref, x2_ref):
    xs = []
    for k in range(2):
        d = dinv_ref[k, :N]
        t = acc_ref[k, :N, :] * d[:, None] + h2_ref[k] * (d * d)[:, None] \
            + b2_ref[...][None, :]
        xs.append(t)
    x1_ref[...] = xs[0]
    x2_ref[...] = xs[1]
    y = _dot(jnp.concatenate(xs, axis=1), fcw_ref[...]) + fcb_ref[...][None, :]
    m = jnp.max(y, axis=1, keepdims=True)
    z = y - m
    out_ref[...] = z - jnp.log(jnp.sum(jnp.exp(z), axis=1, keepdims=True))


_final_call = pl.pallas_call(
    _final_body,
    out_shape=(
        jax.ShapeDtypeStruct((N, C), jnp.float32),
        jax.ShapeDtypeStruct((N, C), jnp.float32),
        jax.ShapeDtypeStruct((N, C), jnp.float32),
    ))


def _pack_edges(edge_index1, edge_index2, edge_weight1, edge_weight2):
    pad = E_PAD - E
    pad_idx = jnp.arange(pad, dtype=jnp.int32) * 37 % N

    def cat(a, fill):
        return jnp.concatenate([a, fill])

    src = jnp.stack([cat(edge_index1[0], pad_idx), cat(edge_index2[0], pad_idx)])
    dst = jnp.stack([cat(edge_index1[1], pad_idx), cat(edge_index2[1], pad_idx)])
    zz = jnp.zeros((pad,), jnp.float32)
    ew = jnp.stack([cat(edge_weight1, zz), cat(edge_weight2, zz)])
    shp = (2, ROWS_TOTAL, LANES)
    return src.reshape(shp), dst.reshape(shp), ew.reshape(shp)


def kernel(x, edge_index1, edge_index2, edge_weight1, edge_weight2,
           W1, b1, W2, b2, fc_W, fc_b):
    src, dst, ew = _pack_edges(edge_index1, edge_index2, edge_weight1, edge_weight2)
    zeros_deg = jnp.zeros((NODE_ROWS_PER_SUB,), jnp.float32)
    zeros_c = jnp.zeros((NODE_ROWS_PER_SUB, C), jnp.float32)

    deg = _deg_call(dst, ew, zeros_deg)
    h1 = _mm_call(x, W1)
    dinv = _dinv_call(deg)
    acc1a = _conv_half(h1[:, :C], src, dst, ew, dinv, zeros_c)
    acc1b = _conv_half(h1[:, C:], src, dst, ew, dinv, zeros_c)
    h2 = _layer1_post(acc1a, acc1b, h1, dinv.reshape(2, 1, N_PAD), b1, W2)
    acc2 = _conv_l2(h2.reshape(2 * N, C), src, dst, ew, dinv, zeros_c)
    out, x_1, x_2 = _final_call(acc2, h2, dinv, b2, fc_W, fc_b)
    return (out, x_1, x_2)


# speedup vs baseline: 35.4660x; 1.7586x over previous
import dataclasses
import functools

import jax
import jax.numpy as jnp
from jax import lax
from jax.experimental import pallas as pl
from jax.experimental.pallas import tpu as pltpu
from jax.experimental.pallas import tpu_sc as plsc

N = 10000
E = 320000
F_IN = 128
H = 128
C = 64

NC = 2
NS = 16
LANES = 128

N_PAD = 10240
ROWS_TOTAL = 2560
E_PAD = ROWS_TOTAL * LANES
ROWS_PER_SUB = ROWS_TOTAL // NS
NODE_ROWS_PER_SUB = N_PAD // NS

_MESH = plsc.VectorSubcoreMesh(core_axis_name="c", subcore_axis_name="s")

_SC_PARAMS = pltpu.CompilerParams()
if "needs_layout_passes" in pltpu.CompilerParams.__dataclass_fields__:
    _SC_PARAMS = dataclasses.replace(
        _SC_PARAMS, needs_layout_passes=False, use_tc_tiling_on_sc=False)


def _deg_body(dst_hbm, ew_hbm, zeros_hbm, out_hbm, dst_v, ew_v, acc):
    c = lax.axis_index("c")
    s = lax.axis_index("s")
    pltpu.sync_copy(zeros_hbm, acc.at[pl.ds(s * NODE_ROWS_PER_SUB, NODE_ROWS_PER_SUB)])
    plsc.subcore_barrier()
    r0 = s * ROWS_PER_SUB
    pltpu.sync_copy(dst_hbm.at[c, pl.ds(r0, ROWS_PER_SUB)], dst_v)
    pltpu.sync_copy(ew_hbm.at[c, pl.ds(r0, ROWS_PER_SUB)], ew_v)

    @pl.loop(0, ROWS_PER_SUB)
    def _(j):
        pltpu.sync_copy(ew_v.at[j], acc.at[dst_v.at[j]], add=True)

    plsc.subcore_barrier()
    sl = pl.ds(s * NODE_ROWS_PER_SUB, NODE_ROWS_PER_SUB)
    pltpu.sync_copy(acc.at[sl], out_hbm.at[c, sl])


_deg_call = pl.kernel(
    _deg_body,
    out_type=jax.ShapeDtypeStruct((NC, N_PAD), jnp.float32),
    mesh=_MESH,
    scratch_types=[
        pltpu.VMEM((ROWS_PER_SUB, LANES), jnp.int32),
        pltpu.VMEM((ROWS_PER_SUB, LANES), jnp.float32),
        pltpu.VMEM_SHARED((N_PAD,), jnp.float32),
    ],
)


def _make_conv(F, stacked):
    KR = 4
    K = KR * LANES
    CH = ROWS_PER_SUB // KR

    def body(table_hbm, src_hbm, dst_hbm, ew_hbm, dinv_hbm, zeros_hbm, out_hbm,
             dinv_v, rows0, rows1, s0, s1, s2, s3, d0, d1, d2, d3,
             e0, e1, e2, e3, acc, sem_i, sem_g0, sem_g1, sem_s):
        c = lax.axis_index("c")
        s = lax.axis_index("s")
        srcb = [s0, s1, s2, s3]
        dstb = [d0, d1, d2, d3]
        ewb = [e0, e1, e2, e3]
        rowsb = [rows0, rows1]
        semg = [sem_g0, sem_g1]
        pltpu.sync_copy(dinv_hbm.at[c], dinv_v)
        pltpu.sync_copy(zeros_hbm, acc.at[pl.ds(s * NODE_ROWS_PER_SUB, NODE_ROWS_PER_SUB), :])
        plsc.subcore_barrier()
        r0 = s * ROWS_PER_SUB
        off = c * N

        def fire_idx(cc, b):
            r = r0 + cc * KR
            pltpu.async_copy(src_hbm.at[c, pl.ds(r, KR)], srcb[b], sem_i)
            pltpu.async_copy(dst_hbm.at[c, pl.ds(r, KR)], dstb[b], sem_i)
            pltpu.async_copy(ew_hbm.at[c, pl.ds(r, KR)], ewb[b], sem_i)

        def wait_idx(cc, b):
            r = r0 + cc * KR
            pltpu.make_async_copy(src_hbm.at[c, pl.ds(r, KR)], srcb[b], sem_i).wait()
            pltpu.make_async_copy(dst_hbm.at[c, pl.ds(r, KR)], dstb[b], sem_i).wait()
            pltpu.make_async_copy(ew_hbm.at[c, pl.ds(r, KR)], ewb[b], sem_i).wait()

        def scompute(b):
            sb, eb = srcb[b], ewb[b]

            @pl.loop(0, KR)
            def _(j):
                for q in range(LANES // 16):
                    sl = (j, pl.ds(q * 16, 16))
                    idx = sb[sl]
                    eb[sl] = eb[sl] * plsc.load_gather(dinv_v, [idx])
                    if stacked:
                        sb[sl] = idx + off

        def fire_gather(b4, b2):
            for j in range(KR):
                pltpu.async_copy(table_hbm.at[srcb[b4].at[j]],
                                 rowsb[b2].at[pl.ds(j * LANES, LANES), :], semg[b2])

        def wait_gather(b4, b2):
            for j in range(KR):
                pltpu.make_async_copy(table_hbm.at[srcb[b4].at[j]],
                                      rowsb[b2].at[pl.ds(j * LANES, LANES), :],
                                      semg[b2]).wait()

        def scale(b4, b2):
            rb, eb = rowsb[b2], ewb[b4]

            @pl.loop(0, KR)
            def _(j):
                base = j * LANES

                @functools.partial(plsc.parallel_loop, 0, LANES // 16, unroll=2)
                def _(g):
                    sv16 = eb[j, pl.ds(g * 16, 16)]
                    i0 = base + g * 16
                    for l in range(16):
                        sv = jnp.take(
                            sv16, jnp.full((16,), l, jnp.int32), axis=0,
                            mode=lax.GatherScatterMode.PROMISE_IN_BOUNDS)
                        for f in range(F // 16):
                            rb[i0 + l, pl.ds(f * 16, 16)] = \
                                rb[i0 + l, pl.ds(f * 16, 16)] * sv

        def fire_scatter(b4, b2):
            for j in range(KR):
                pltpu.async_copy(rowsb[b2].at[pl.ds(j * LANES, LANES), :],
                                 acc.at[dstb[b4].at[j]], sem_s, add=True)

        def wait_scatter(b4, b2):
            for j in range(KR):
                pltpu.make_async_copy(rowsb[b2].at[pl.ds(j * LANES, LANES), :],
                                      acc.at[dstb[b4].at[j]], sem_s).wait()

        fire_idx(0, 0)
        wait_idx(0, 0)
        scompute(0)
        fire_gather(0, 0)
        fire_idx(1, 1)
        wait_idx(1, 1)
        scompute(1)
        fire_idx(2, 2)

        @pl.loop(0, CH, step=4)
        def _(ci):
            for k in range(4):
                cc = ci + k
                b4 = k % 4
                b2 = k % 2

                @pl.when(cc >= 1)
                def _():
                    wait_scatter((k - 1) % 4, (k - 1) % 2)

                @pl.when(cc + 1 < CH)
                def _():
                    fire_gather((k + 1) % 4, (k + 1) % 2)

                @pl.when(cc + 2 < CH)
                def _():
                    wait_idx(cc + 2, (k + 2) % 4)
                    scompute((k + 2) % 4)

                @pl.when(cc + 3 < CH)
                def _():
                    fire_idx(cc + 3, (k + 3) % 4)

                wait_gather(b4, b2)
                scale(b4, b2)
                fire_scatter(b4, b2)

        wait_scatter((CH - 1) % 4, (CH - 1) % 2)
        plsc.subcore_barrier()
        sl = pl.ds(s * NODE_ROWS_PER_SUB, NODE_ROWS_PER_SUB)
        pltpu.sync_copy(acc.at[sl, :], out_hbm.at[c, sl, :])

    idx_bufs = ([pltpu.VMEM((KR, LANES), jnp.int32)] * 8
                + [pltpu.VMEM((KR, LANES), jnp.float32)] * 4)
    return pl.kernel(
        body,
        out_type=jax.ShapeDtypeStruct((NC, N_PAD, F), jnp.float32),
        mesh=_MESH,
        scratch_types=[
            pltpu.VMEM((N_PAD,), jnp.float32),
            pltpu.VMEM((K, F), jnp.float32),
            pltpu.VMEM((K, F), jnp.float32),
            *idx_bufs,
            pltpu.VMEM_SHARED((N_PAD, F), jnp.float32),
            pltpu.SemaphoreType.DMA,
            pltpu.SemaphoreType.DMA,
            pltpu.SemaphoreType.DMA,
            pltpu.SemaphoreType.DMA,
        ],
        compiler_params=_SC_PARAMS,
    )


_conv_half = _make_conv(C, stacked=False)
_conv_l2 = _make_conv(C, stacked=True)


def _dot(a, b):
    return lax.dot_general(a, b, (((1,), (0,)), ((), ())),
                           preferred_element_type=jnp.float32,
                           precision=lax.Precision.HIGHEST)


def _mm_body(x_ref, w_ref, o_ref):
    o_ref[...] = _dot(x_ref[...], w_ref[...])


_mm_call = pl.pallas_call(
    _mm_body, out_shape=jax.ShapeDtypeStruct((N, H), jnp.float32))


def _dinv_body(deg_ref, o_ref):
    o_ref[...] = lax.rsqrt(deg_ref[...] + 1.0)


_dinv_call = pl.pallas_call(
    _dinv_body, out_shape=jax.ShapeDtypeStruct((NC, N_PAD), jnp.float32))


def _layer1_post_body(acca_ref, accb_ref, h1_ref, dinv_ref, b1_ref, w2_ref, o_ref):
    d = dinv_ref[0, 0, :N]
    a = jnp.concatenate([acca_ref[0, :N, :], accb_ref[0, :N, :]], axis=1)
    t = a * d[:, None] + h1_ref[...] * (d * d)[:, None] + b1_ref[...][None, :]
    o_ref[0, :, :] = _dot(jnp.maximum(t, 0.0), w2_ref[...])


_layer1_post = pl.pallas_call(
    _layer1_post_body,
    grid=(2,),
    in_specs=[
        pl.BlockSpec((1, N_PAD, C), lambda k: (k, 0, 0)),
        pl.BlockSpec((1, N_PAD, C), lambda k: (k, 0, 0)),
        pl.BlockSpec((N, H), lambda k: (0, 0)),
        pl.BlockSpec((1, 1, N_PAD), lambda k: (k, 0, 0)),
        pl.BlockSpec((H,), lambda k: (0,)),
        pl.BlockSpec((H, C), lambda k: (0, 0)),
    ],
    out_specs=pl.BlockSpec((1, N, C), lambda k: (k, 0, 0)),
    out_shape=jax.ShapeDtypeStruct((2, N, C), jnp.float32))


def _final_body(acc_ref, h2_ref, dinv_ref, b2_ref, fcw_ref, fcb_ref,
                out_ref, x1_ref, x2_ref):
    xs = []
    for k in range(2):
        d = dinv_ref[k, :N]
        t = acc_ref[k, :N, :] * d[:, None] + h2_ref[k] * (d * d)[:, None] \
            + b2_ref[...][None, :]
        xs.append(t)
    x1_ref[...] = xs[0]
    x2_ref[...] = xs[1]
    y = _dot(jnp.concatenate(xs, axis=1), fcw_ref[...]) + fcb_ref[...][None, :]
    m = jnp.max(y, axis=1, keepdims=True)
    z = y - m
    out_ref[...] = z - jnp.log(jnp.sum(jnp.exp(z), axis=1, keepdims=True))


_final_call = pl.pallas_call(
    _final_body,
    out_shape=(
        jax.ShapeDtypeStruct((N, C), jnp.float32),
        jax.ShapeDtypeStruct((N, C), jnp.float32),
        jax.ShapeDtypeStruct((N, C), jnp.float32),
    ))


def _pack_edges(edge_index1, edge_index2, edge_weight1, edge_weight2):
    pad = E_PAD - E
    pad_idx = jnp.arange(pad, dtype=jnp.int32) * 37 % N

    def cat(a, fill):
        return jnp.concatenate([a, fill])

    src = jnp.stack([cat(edge_index1[0], pad_idx), cat(edge_index2[0], pad_idx)])
    dst = jnp.stack([cat(edge_index1[1], pad_idx), cat(edge_index2[1], pad_idx)])
    zz = jnp.zeros((pad,), jnp.float32)
    ew = jnp.stack([cat(edge_weight1, zz), cat(edge_weight2, zz)])
    shp = (2, ROWS_TOTAL, LANES)
    return src.reshape(shp), dst.reshape(shp), ew.reshape(shp)


def kernel(x, edge_index1, edge_index2, edge_weight1, edge_weight2,
           W1, b1, W2, b2, fc_W, fc_b):
    src, dst, ew = _pack_edges(edge_index1, edge_index2, edge_weight1, edge_weight2)
    zeros_deg = jnp.zeros((NODE_ROWS_PER_SUB,), jnp.float32)
    zeros_c = jnp.zeros((NODE_ROWS_PER_SUB, C), jnp.float32)

    deg = _deg_call(dst, ew, zeros_deg)
    h1 = _mm_call(x, W1)
    dinv = _dinv_call(deg)
    acc1a = _conv_half(h1[:, :C], src, dst, ew, dinv, zeros_c)
    acc1b = _conv_half(h1[:, C:], src, dst, ew, dinv, zeros_c)
    h2 = _layer1_post(acc1a, acc1b, h1, dinv.reshape(2, 1, N_PAD), b1, W2)
    acc2 = _conv_l2(h2.reshape(2 * N, C), src, dst, ew, dinv, zeros_c)
    out, x_1, x_2 = _final_call(acc2, h2, dinv, b2, fc_W, fc_b)
    return (out, x_1, x_2)
